# baseline (speedup 1.0000x reference)
import functools

import numpy as np
import jax
import jax.numpy as jnp
from jax import lax
from jax.experimental import pallas as pl
from jax.experimental.pallas import tpu as pltpu
from jax.experimental.pallas import tpu_sc as plsc

D = 128
LG = D // 16
CHUNK = 128
NBUF = 4


def _tc_addend(pos_table, tt_f32, type_table):
    t = pos_table.shape[0]
    blk = 2048

    def body(pos_ref, tt_ref, ty_ref, o_ref):
        t0 = ty_ref[0:1, :]
        t1 = ty_ref[1:2, :]
        o_ref[...] = pos_ref[...] + t0 + tt_ref[...] * (t1 - t0)

    return pl.pallas_call(
        body,
        grid=(t // blk,),
        in_specs=[
            pl.BlockSpec((blk, D), lambda i: (i, 0)),
            pl.BlockSpec((blk, 1), lambda i: (i, 0)),
            pl.BlockSpec((2, D), lambda i: (0, 0)),
        ],
        out_specs=pl.BlockSpec((blk, D), lambda i: (i, 0)),
        out_shape=jax.ShapeDtypeStruct((t, D), jnp.float32),
    )(pos_table, tt_f32, type_table)


def _xlane_sum(v):
    lanes = lax.iota(jnp.int32, 16)
    dn = lax.GatherDimensionNumbers(offset_dims=(), collapsed_slice_dims=(0,),
                                    start_index_map=(0,))
    for k in range(4):
        idx = lanes ^ (1 << k)
        v = v + lax.gather(v, idx[:, None], dimension_numbers=dn,
                           slice_sizes=(1,),
                           mode=lax.GatherScatterMode.PROMISE_IN_BOUNDS)
    return v


def _rsqrt_nr(v):
    half = v * jnp.float32(0.5)
    i = lax.bitcast_convert_type(v, jnp.int32)
    i = jnp.int32(0x5F3759DF) - lax.shift_right_logical(i, 1)
    y = lax.bitcast_convert_type(i, jnp.float32)
    for _ in range(3):
        y = y * (jnp.float32(1.5) - half * y * y)
    return y


def _sc_fused(ids_t2d, oidx2d, word_table, addend, gam, bet, b):
    n_chunks_total, chunk = ids_t2d.shape
    ntok = n_chunks_total * chunk
    info = plsc.get_sparse_core_info()
    nc, ns = info.num_cores, info.num_subcores
    nw = nc * ns
    chunks_per_w = n_chunks_total // nw
    tok_per_chunk = chunk // b
    inv_d = jnp.float32(1.0 / D)
    eps = jnp.float32(1e-12)
    lb = b.bit_length() - 1

    mesh = plsc.VectorSubcoreMesh(core_axis_name="c", subcore_axis_name="s")

    @functools.partial(
        pl.kernel,
        mesh=mesh,
        out_type=jax.ShapeDtypeStruct((ntok, D), jnp.float32),
        scratch_types=(
            [pltpu.VMEM((chunks_per_w, chunk), jnp.int32)] * 2
            + [pltpu.VMEM((chunk, D), jnp.float32)] * NBUF
            + [pltpu.VMEM((tok_per_chunk, D), jnp.float32)] * NBUF
            + [pltpu.VMEM((2, LG, 16), jnp.float32)]
            + [pltpu.SemaphoreType.DMA] * (3 * NBUF)
        ),
    )
    def fused_k(ids_hbm, oidx_hbm, table_hbm, add_hbm, gb_hbm, out_hbm,
                idx_v, oidx_v, *rest):
        gbuf = rest[0:NBUF]
        abuf = rest[NBUF:2 * NBUF]
        gb_v = rest[2 * NBUF]
        sg = rest[2 * NBUF + 1:2 * NBUF + 1 + NBUF]
        sa = rest[2 * NBUF + 1 + NBUF:2 * NBUF + 1 + 2 * NBUF]
        so = rest[2 * NBUF + 1 + 2 * NBUF:2 * NBUF + 1 + 3 * NBUF]
        wid = lax.axis_index("s") * nc + lax.axis_index("c")
        base_chunk = wid * chunks_per_w
        base_tok = wid * chunks_per_w * tok_per_chunk
        pltpu.sync_copy(ids_hbm.at[pl.ds(base_chunk, chunks_per_w)], idx_v)
        pltpu.sync_copy(oidx_hbm.at[pl.ds(base_chunk, chunks_per_w)], oidx_v)
        pltpu.sync_copy(gb_hbm, gb_v)
        gamv = [gb_v[0, k] for k in range(LG)]
        betv = [gb_v[1, k] for k in range(LG)]

        def start_in(j):
            s = j % NBUF
            g = pltpu.async_copy(table_hbm.at[idx_v.at[j]], gbuf[s], sg[s])
            a = pltpu.async_copy(
                add_hbm.at[pl.ds(base_tok + j * tok_per_chunk, tok_per_chunk)],
                abuf[s], sa[s])
            return g, a

        inflight = [None] * NBUF
        outflight = [None] * NBUF
        prime = min(NBUF - 1, chunks_per_w)
        for j in range(prime):
            inflight[j % NBUF] = start_in(j)

        for j in range(chunks_per_w):
            s = j % NBUF
            gcp, acp = inflight[s]
            gcp.wait()
            acp.wait()
            G, A = gbuf[s], abuf[s]

            def row_body(r, carry, G=G, A=A):
                ar = lax.shift_right_logical(r, lb)
                x = [G[r, pl.ds(16 * k, 16)] + A[ar, pl.ds(16 * k, 16)]
                     for k in range(LG)]
                s0 = ((x[0] + x[1]) + (x[2] + x[3])) + \
                     ((x[4] + x[5]) + (x[6] + x[7]))
                mean = _xlane_sum(s0) * inv_d
                xc = [x[k] - mean for k in range(LG)]
                sq = [xc[k] * xc[k] for k in range(LG)]
                s1 = ((sq[0] + sq[1]) + (sq[2] + sq[3])) + \
                     ((sq[4] + sq[5]) + (sq[6] + sq[7]))
                var = _xlane_sum(s1) * inv_d
                rstd = _rsqrt_nr(var + eps)
                for k in range(LG):
                    G[r, pl.ds(16 * k, 16)] = xc[k] * (rstd * gamv[k]) + betv[k]
                return carry

            lax.fori_loop(0, chunk, row_body, 0)

            outflight[s] = pltpu.async_copy(G, out_hbm.at[oidx_v.at[j]], so[s])
            nxt = j + prime
            if nxt < chunks_per_w:
                ps = nxt % NBUF
                if outflight[ps] is not None:
                    outflight[ps].wait()
                    outflight[ps] = None
                inflight[ps] = start_in(nxt)

        for s in range(NBUF):
            if outflight[s] is not None:
                outflight[s].wait()

    return fused_k(ids_t2d, oidx2d, word_table, addend, jnp.stack([gam, bet]))


def kernel(ids, token_type_ids, word_table, pos_table, type_table, ln_gamma, ln_beta):
    b, t = ids.shape
    ids_t2d = ids.astype(jnp.int32).T.reshape(-1, CHUNK)
    i = np.arange(b * t)
    oidx = ((i % b) * t + i // b).astype(np.int32).reshape(-1, CHUNK)
    tt_f32 = token_type_ids.astype(jnp.float32).reshape(t, 1)
    addend = _tc_addend(pos_table, tt_f32, type_table)
    out = _sc_fused(ids_t2d, jnp.asarray(oidx), word_table, addend,
                    ln_gamma.reshape(LG, 16), ln_beta.reshape(LG, 16), b)
    return out.reshape(b, t, D)


# speedup vs baseline: 1.1143x; 1.1143x over previous
import functools

import numpy as np
import jax
import jax.numpy as jnp
from jax import lax
from jax.experimental import pallas as pl
from jax.experimental.pallas import tpu as pltpu
from jax.experimental.pallas import tpu_sc as plsc

D = 128
LG = D // 16
CHUNK = 128
NBUF = 4


def _tc_addend(pos_table, tt_f32, type_table):
    t = pos_table.shape[0]
    blk = 2048

    def body(pos_ref, tt_ref, ty_ref, o_ref):
        t0 = ty_ref[0:1, :]
        t1 = ty_ref[1:2, :]
        o_ref[...] = pos_ref[...] + t0 + tt_ref[...] * (t1 - t0)

    return pl.pallas_call(
        body,
        grid=(t // blk,),
        in_specs=[
            pl.BlockSpec((blk, D), lambda i: (i, 0)),
            pl.BlockSpec((blk, 1), lambda i: (i, 0)),
            pl.BlockSpec((2, D), lambda i: (0, 0)),
        ],
        out_specs=pl.BlockSpec((blk, D), lambda i: (i, 0)),
        out_shape=jax.ShapeDtypeStruct((t, D), jnp.float32),
    )(pos_table, tt_f32, type_table)


def _xlane_sum(v):
    lanes = lax.iota(jnp.int32, 16)
    dn = lax.GatherDimensionNumbers(offset_dims=(), collapsed_slice_dims=(0,),
                                    start_index_map=(0,))
    for k in range(4):
        idx = lanes ^ (1 << k)
        v = v + lax.gather(v, idx[:, None], dimension_numbers=dn,
                           slice_sizes=(1,),
                           mode=lax.GatherScatterMode.PROMISE_IN_BOUNDS)
    return v


def _rsqrt_nr(v):
    half = v * jnp.float32(0.5)
    i = lax.bitcast_convert_type(v, jnp.int32)
    i = jnp.int32(0x5F3759DF) - lax.shift_right_logical(i, 1)
    y = lax.bitcast_convert_type(i, jnp.float32)
    for _ in range(2):
        y = y * (jnp.float32(1.5) - half * y * y)
    return y


def _sc_fused(ids_t2d, oidx2d, word_table, addend, gam, bet, b):
    n_chunks_total, chunk = ids_t2d.shape
    ntok = n_chunks_total * chunk
    info = plsc.get_sparse_core_info()
    nc, ns = info.num_cores, info.num_subcores
    nw = nc * ns
    chunks_per_w = n_chunks_total // nw
    tok_per_chunk = chunk // b
    inv_d = jnp.float32(1.0 / D)
    eps = jnp.float32(1e-12)
    lb = b.bit_length() - 1

    mesh = plsc.VectorSubcoreMesh(core_axis_name="c", subcore_axis_name="s")

    @functools.partial(
        pl.kernel,
        mesh=mesh,
        out_type=jax.ShapeDtypeStruct((ntok, D), jnp.float32),
        scratch_types=(
            [pltpu.VMEM((chunks_per_w, chunk), jnp.int32)] * 2
            + [pltpu.VMEM((chunk, D), jnp.float32)] * NBUF
            + [pltpu.VMEM((tok_per_chunk, D), jnp.float32)] * NBUF
            + [pltpu.VMEM((2, LG, 16), jnp.float32)]
            + [pltpu.SemaphoreType.DMA] * (3 * NBUF)
        ),
    )
    def fused_k(ids_hbm, oidx_hbm, table_hbm, add_hbm, gb_hbm, out_hbm,
                idx_v, oidx_v, *rest):
        gbuf = rest[0:NBUF]
        abuf = rest[NBUF:2 * NBUF]
        gb_v = rest[2 * NBUF]
        sg = rest[2 * NBUF + 1:2 * NBUF + 1 + NBUF]
        sa = rest[2 * NBUF + 1 + NBUF:2 * NBUF + 1 + 2 * NBUF]
        so = rest[2 * NBUF + 1 + 2 * NBUF:2 * NBUF + 1 + 3 * NBUF]
        wid = lax.axis_index("s") * nc + lax.axis_index("c")
        base_chunk = wid * chunks_per_w
        base_tok = wid * chunks_per_w * tok_per_chunk
        pltpu.sync_copy(ids_hbm.at[pl.ds(base_chunk, chunks_per_w)], idx_v)
        pltpu.sync_copy(oidx_hbm.at[pl.ds(base_chunk, chunks_per_w)], oidx_v)
        pltpu.sync_copy(gb_hbm, gb_v)
        gamv = [gb_v[0, k] for k in range(LG)]
        betv = [gb_v[1, k] for k in range(LG)]

        def start_in(j):
            s = j % NBUF
            g = pltpu.async_copy(table_hbm.at[idx_v.at[j]], gbuf[s], sg[s])
            a = pltpu.async_copy(
                add_hbm.at[pl.ds(base_tok + j * tok_per_chunk, tok_per_chunk)],
                abuf[s], sa[s])
            return g, a

        inflight = [None] * NBUF
        outflight = [None] * NBUF
        prime = min(NBUF - 1, chunks_per_w)
        for j in range(prime):
            inflight[j % NBUF] = start_in(j)

        for j in range(chunks_per_w):
            s = j % NBUF
            gcp, acp = inflight[s]
            gcp.wait()
            acp.wait()
            G, A = gbuf[s], abuf[s]

            @plsc.parallel_loop(0, tok_per_chunk, unroll=2)
            def _(tk, G=G, A=A):
                a = [A[tk, pl.ds(16 * k, 16)] for k in range(LG)]
                for bb in range(b):
                    r = tk * b + bb
                    g = [G[r, pl.ds(16 * k, 16)] for k in range(LG)]
                    x = [g[k] + a[k] for k in range(LG)]
                    s0 = ((x[0] + x[1]) + (x[2] + x[3])) + \
                         ((x[4] + x[5]) + (x[6] + x[7]))
                    mean = _xlane_sum(s0) * inv_d
                    xc = [x[k] - mean for k in range(LG)]
                    sq = [xc[k] * xc[k] for k in range(LG)]
                    s1 = ((sq[0] + sq[1]) + (sq[2] + sq[3])) + \
                         ((sq[4] + sq[5]) + (sq[6] + sq[7]))
                    var = _xlane_sum(s1) * inv_d
                    rstd = _rsqrt_nr(var + eps)
                    for k in range(LG):
                        G[r, pl.ds(16 * k, 16)] = \
                            xc[k] * (rstd * gamv[k]) + betv[k]

            outflight[s] = pltpu.async_copy(G, out_hbm.at[oidx_v.at[j]], so[s])
            nxt = j + prime
            if nxt < chunks_per_w:
                ps = nxt % NBUF
                if outflight[ps] is not None:
                    outflight[ps].wait()
                    outflight[ps] = None
                inflight[ps] = start_in(nxt)

        for s in range(NBUF):
            if outflight[s] is not None:
                outflight[s].wait()

    return fused_k(ids_t2d, oidx2d, word_table, addend, jnp.stack([gam, bet]))


def kernel(ids, token_type_ids, word_table, pos_table, type_table, ln_gamma, ln_beta):
    b, t = ids.shape
    ids_t2d = ids.astype(jnp.int32).T.reshape(-1, CHUNK)
    i = np.arange(b * t)
    oidx = ((i % b) * t + i // b).astype(np.int32).reshape(-1, CHUNK)
    tt_f32 = token_type_ids.astype(jnp.float32).reshape(t, 1)
    addend = _tc_addend(pos_table, tt_f32, type_table)
    out = _sc_fused(ids_t2d, jnp.asarray(oidx), word_table, addend,
                    ln_gamma.reshape(LG, 16), ln_beta.reshape(LG, 16), b)
    return out.reshape(b, t, D)
